# baseline (speedup 1.0000x reference)
import jax
import jax.numpy as jnp
from jax import lax
from jax.experimental import pallas as pl
from jax.experimental.pallas import tpu as pltpu
from jax.experimental.pallas import tpu_sc as plsc

MEMORY_UNIT = 256
MAX_MEMORY = 100000
OUT_DIM = 512
UPDATE_SIZE = 3 + MEMORY_UNIT
Y_DIM = OUT_DIM - UPDATE_SIZE
JUMP_THRESHOLD = 0.5
MIN_SIM_TO_JUMP = 0.5

ROWS_PER_BLOCK = 10000
NUM_BLOCKS = MAX_MEMORY // ROWS_PER_BLOCK

SC_CHUNK = 800
SC_NUM_CHUNKS = MAX_MEMORY // SC_CHUNK
NW = 32


def _controller_kernel(xj_ref, w_mat_ref, b_ref, out_ref):
    out_ref[...] = (
        jax.lax.dot_general(
            xj_ref[...], w_mat_ref[...], (((1,), (1,)), ((), ())),
            preferred_element_type=jnp.float32,
            precision=jax.lax.Precision.HIGHEST,
        )
        + b_ref[...]
    )


def _sc_copy(mem_hbm, m_hbm, sjw_hbm, mem2_hbm, sjw_v):
    wid = lax.axis_index("s") * 2 + lax.axis_index("c")
    pltpu.sync_copy(sjw_hbm, sjw_v)
    w = sjw_v[...][15]
    nk = jnp.where(wid < SC_NUM_CHUNKS % NW, SC_NUM_CHUNKS // NW + 1,
                   SC_NUM_CHUNKS // NW)

    def chunk_body(k, carry):
        base = (wid + NW * k) * SC_CHUNK
        pltpu.sync_copy(mem_hbm.at[pl.ds(base, SC_CHUNK)],
                        mem2_hbm.at[pl.ds(base, SC_CHUNK)])
        return carry

    lax.fori_loop(0, nk, chunk_body, 0)

    @pl.when((wid == 0) & (w > 0.5))
    def _():
        pltpu.sync_copy(m_hbm, mem2_hbm.at[pl.ds(0, 1)])


def _scan_kernel(ctrl_ref, mem_ref, mem_any_ref,
                 read_ref, land_ref, best_ref, pos_ref, sem):
    i = pl.program_id(0)

    @pl.when(i == 0)
    def _():
        best_ref[0] = -jnp.inf
        pos_ref[0] = 0

    w = ctrl_ref[0, Y_DIM + 2]
    m_row = ctrl_ref[0:1, Y_DIM + 3:]

    blk = mem_ref[...]
    d = blk - m_row
    d2 = jnp.sum(d * d, axis=1, keepdims=True)
    sims = 1.0 - jnp.sqrt(d2) * (1.0 / MEMORY_UNIT)
    local_best = jnp.max(sims)
    rows = jax.lax.broadcasted_iota(jnp.int32, (ROWS_PER_BLOCK, 1), 0)
    local_pos = jnp.min(
        jnp.where(sims == local_best, rows, ROWS_PER_BLOCK)
    ) + i * ROWS_PER_BLOCK

    better = local_best > best_ref[0]
    best_ref[0] = jnp.where(better, local_best, best_ref[0])
    pos_ref[0] = jnp.where(better, local_pos, pos_ref[0])

    @pl.when(i == NUM_BLOCKS - 1)
    def _():
        s = ctrl_ref[0, Y_DIM]
        j = ctrl_ref[0, Y_DIM + 1]
        jumped = jnp.where(
            w > 0.5, 0,
            jnp.where(best_ref[0] > MIN_SIM_TO_JUMP, pos_ref[0], 0),
        )
        head0 = jnp.where(j > JUMP_THRESHOLD, jumped, 0)
        shift = jnp.floor(s * 3.0 - 1e-9).astype(jnp.int32) - 1
        head = jnp.mod(head0 + shift, MAX_MEMORY)
        copy = pltpu.make_async_copy(
            mem_any_ref.at[pl.ds(head, 1)], land_ref, sem)
        copy.start()
        copy.wait()
        read_ref[...] = jnp.where(
            (head == 0) & (w > 0.5), m_row, land_ref[...])


def kernel(x, W, b, memory, previous_read, interpret=False):
    xj = jnp.concatenate([x, previous_read[None, :]], axis=1)

    out = pl.pallas_call(
        _controller_kernel,
        out_shape=jax.ShapeDtypeStruct((1, OUT_DIM), jnp.float32),
        interpret=interpret,
    )(xj, W, b[None, :])

    y = out[0, :Y_DIM]
    m_2d = out[0:1, Y_DIM + 3:]
    sjw = out[0, Y_DIM - 13:Y_DIM + 3]

    sc_fn = pl.kernel(
        _sc_copy,
        mesh=plsc.VectorSubcoreMesh(core_axis_name="c", subcore_axis_name="s"),
        out_type=jax.ShapeDtypeStruct((MAX_MEMORY, MEMORY_UNIT), jnp.float32),
        scratch_types=[
            pltpu.VMEM((16,), jnp.float32),
        ],
        compiler_params=pltpu.CompilerParams(needs_layout_passes=False),
    )
    mem2 = sc_fn(memory, m_2d, sjw)

    read = pl.pallas_call(
        _scan_kernel,
        grid=(NUM_BLOCKS,),
        in_specs=[
            pl.BlockSpec((1, OUT_DIM), lambda i: (0, 0)),
            pl.BlockSpec((ROWS_PER_BLOCK, MEMORY_UNIT), lambda i: (i, 0)),
            pl.BlockSpec(memory_space=pl.ANY),
        ],
        out_specs=pl.BlockSpec((1, MEMORY_UNIT), lambda i: (0, 0)),
        out_shape=jax.ShapeDtypeStruct((1, MEMORY_UNIT), jnp.float32),
        scratch_shapes=[
            pltpu.VMEM((1, MEMORY_UNIT), jnp.float32),
            pltpu.SMEM((1,), jnp.float32),
            pltpu.SMEM((1,), jnp.int32),
            pltpu.SemaphoreType.DMA,
        ],
        compiler_params=pltpu.CompilerParams(
            dimension_semantics=("arbitrary",),
        ),
        interpret=interpret,
    )(out, memory, memory)

    return y, read[0], mem2


# speedup vs baseline: 41.1524x; 41.1524x over previous
import jax
import jax.numpy as jnp
from jax.experimental import pallas as pl
from jax.experimental.pallas import tpu as pltpu

MEMORY_UNIT = 256
MAX_MEMORY = 100000
OUT_DIM = 512
UPDATE_SIZE = 3 + MEMORY_UNIT
Y_DIM = OUT_DIM - UPDATE_SIZE
JUMP_THRESHOLD = 0.5
MIN_SIM_TO_JUMP = 0.5

ROWS_PER_BLOCK = 10000
NUM_BLOCKS = MAX_MEMORY // ROWS_PER_BLOCK


def _ntm_kernel(xj_ref, w_mat_ref, b_ref, mem_ref, mem_any_ref,
                y_ref, read_ref, mem2_ref,
                ctrl_ref, land_ref, best_ref, pos_ref, sem):
    i = pl.program_id(0)

    @pl.when(i == 0)
    def _():
        out = jax.lax.dot_general(
            xj_ref[...], w_mat_ref[...], (((1,), (1,)), ((), ())),
            preferred_element_type=jnp.float32,
            precision=jax.lax.Precision.HIGHEST,
        ) + b_ref[...]
        ctrl_ref[...] = out
        y_ref[...] = out[:, :Y_DIM]
        best_ref[0] = -jnp.inf
        pos_ref[0] = 0

    w = ctrl_ref[0, Y_DIM + 2]
    m_row = ctrl_ref[0:1, Y_DIM + 3:]

    blk = mem_ref[...]
    mem2_ref[...] = blk

    @pl.when((i == 0) & (w > 0.5))
    def _():
        mem2_ref[0:1, :] = m_row

    d = blk - m_row
    d2 = jnp.sum(d * d, axis=1, keepdims=True)
    sims = 1.0 - jnp.sqrt(d2) * (1.0 / MEMORY_UNIT)
    local_best = jnp.max(sims)
    rows = jax.lax.broadcasted_iota(jnp.int32, (ROWS_PER_BLOCK, 1), 0)
    local_pos = jnp.min(
        jnp.where(sims == local_best, rows, ROWS_PER_BLOCK)
    ) + i * ROWS_PER_BLOCK

    better = local_best > best_ref[0]
    best_ref[0] = jnp.where(better, local_best, best_ref[0])
    pos_ref[0] = jnp.where(better, local_pos, pos_ref[0])

    @pl.when(i == NUM_BLOCKS - 1)
    def _():
        s = ctrl_ref[0, Y_DIM]
        j = ctrl_ref[0, Y_DIM + 1]
        jumped = jnp.where(
            w > 0.5, 0,
            jnp.where(best_ref[0] > MIN_SIM_TO_JUMP, pos_ref[0], 0),
        )
        head0 = jnp.where(j > JUMP_THRESHOLD, jumped, 0)
        shift = jnp.floor(s * 3.0 - 1e-9).astype(jnp.int32) - 1
        head = jnp.mod(head0 + shift, MAX_MEMORY)
        copy = pltpu.make_async_copy(
            mem_any_ref.at[pl.ds(head, 1)], land_ref, sem)
        copy.start()
        copy.wait()
        read_ref[...] = jnp.where(
            (head == 0) & (w > 0.5), m_row, land_ref[...])


def kernel(x, W, b, memory, previous_read, interpret=False):
    xj = jnp.concatenate([x, previous_read[None, :]], axis=1)

    y, read, mem2 = pl.pallas_call(
        _ntm_kernel,
        grid=(NUM_BLOCKS,),
        in_specs=[
            pl.BlockSpec((1, OUT_DIM), lambda i: (0, 0)),
            pl.BlockSpec((OUT_DIM, OUT_DIM), lambda i: (0, 0)),
            pl.BlockSpec((1, OUT_DIM), lambda i: (0, 0)),
            pl.BlockSpec((ROWS_PER_BLOCK, MEMORY_UNIT), lambda i: (i, 0)),
            pl.BlockSpec(memory_space=pl.ANY),
        ],
        out_specs=[
            pl.BlockSpec((1, Y_DIM), lambda i: (0, 0)),
            pl.BlockSpec((1, MEMORY_UNIT), lambda i: (0, 0)),
            pl.BlockSpec((ROWS_PER_BLOCK, MEMORY_UNIT), lambda i: (i, 0)),
        ],
        out_shape=[
            jax.ShapeDtypeStruct((1, Y_DIM), jnp.float32),
            jax.ShapeDtypeStruct((1, MEMORY_UNIT), jnp.float32),
            jax.ShapeDtypeStruct((MAX_MEMORY, MEMORY_UNIT), jnp.float32),
        ],
        scratch_shapes=[
            pltpu.VMEM((1, OUT_DIM), jnp.float32),
            pltpu.VMEM((1, MEMORY_UNIT), jnp.float32),
            pltpu.SMEM((1,), jnp.float32),
            pltpu.SMEM((1,), jnp.int32),
            pltpu.SemaphoreType.DMA,
        ],
        compiler_params=pltpu.CompilerParams(
            dimension_semantics=("arbitrary",),
        ),
        interpret=interpret,
    )(xj, W, b[None, :], memory, memory)

    return y[0], read[0], mem2
